# baseline (speedup 1.0000x reference)
import jax
import jax.numpy as jnp
from jax.experimental import pallas as pl
from jax.experimental.pallas import tpu as pltpu


def _tc_body(start_ref, pe_ref, x_ref, o_ref):
    S = x_ref.shape[0]
    row0 = pl.multiple_of(start_ref[0], 8)
    o_ref[...] = x_ref[...] + pe_ref[pl.ds(row0, S), :]


import functools


@functools.partial(jax.jit, static_argnums=(3,))
def _tc_pe_add(x2d, position_embedding, start, batch):
    N, D = x2d.shape
    S = N // batch
    return pl.pallas_call(
        _tc_body,
        grid_spec=pltpu.PrefetchScalarGridSpec(
            num_scalar_prefetch=1,
            grid=(batch,),
            in_specs=[
                pl.BlockSpec(position_embedding.shape, lambda i, s_ref: (0, 0)),
                pl.BlockSpec((S, D), lambda i, s_ref: (i, 0)),
            ],
            out_specs=pl.BlockSpec((S, D), lambda i, s_ref: (i, 0)),
        ),
        out_shape=jax.ShapeDtypeStruct(x2d.shape, x2d.dtype),
        compiler_params=pltpu.CompilerParams(
            dimension_semantics=("arbitrary",),
        ),
    )(start, position_embedding, x2d)


def kernel(x, position_embedding, position_start):
    B, S, D = x.shape
    start = jnp.asarray(position_start, jnp.int32).reshape((1,))
    out2d = _tc_pe_add(x.reshape(B * S, D), position_embedding, start, B)
    return out2d.reshape(B, S, D)


# speedup vs baseline: 1.0296x; 1.0296x over previous
import functools

import jax
import jax.numpy as jnp
from jax.experimental import pallas as pl
from jax.experimental.pallas import tpu as pltpu


def _tc_body(pe_ref, x_ref, o_ref):
    o_ref[...] = x_ref[...] + pe_ref[...]


@functools.partial(jax.jit, static_argnums=(2,))
def _tc_pe_add(x2d, pe, batch):
    N, D = x2d.shape
    S = N // batch
    return pl.pallas_call(
        _tc_body,
        grid=(batch,),
        in_specs=[
            pl.BlockSpec(pe.shape, lambda i: (0, 0)),
            pl.BlockSpec((S, D), lambda i: (i, 0)),
        ],
        out_specs=pl.BlockSpec((S, D), lambda i: (i, 0)),
        out_shape=jax.ShapeDtypeStruct(x2d.shape, x2d.dtype),
        compiler_params=pltpu.CompilerParams(
            dimension_semantics=("parallel",),
        ),
    )(pe, x2d)


def kernel(x, position_embedding, position_start):
    B, S, D = x.shape
    M = position_embedding.shape[0]
    pe = jax.lax.dynamic_slice(
        position_embedding,
        (jnp.asarray(position_start, jnp.int32), jnp.int32(0)),
        (S, D),
    ) if M != S else position_embedding
    out2d = _tc_pe_add(x.reshape(B * S, D), pe, B)
    return out2d.reshape(B, S, D)
